# baseline (speedup 1.0000x reference)
import functools

import jax
import jax.numpy as jnp
import numpy as np
from jax import lax
from jax.experimental import pallas as pl
from jax.experimental.pallas import tpu as pltpu
from jax.experimental.pallas import tpu_sc as plsc

N = 10000
E = 160000
D = 128
H = 4
R = 4
LOOP_LAMBDA = 0.5
NEG_SLOPE = 0.2

NCORE = 2
NSUB = 16
NW = NCORE * NSUB
L = 16

NP = 10240
EW = 5120
EPAD = NW * EW
DUMMY_DST = N + 200

_TPERM = np.arange(D).reshape(D // 32, 2, 16).transpose(0, 2, 1).reshape(-1)

_mesh = plsc.VectorSubcoreMesh(
    core_axis_name="c", subcore_axis_name="s", num_cores=NCORE,
    num_subcores=NSUB)
_sc_params = pltpu.CompilerParams(needs_layout_passes=False)


def _wid():
    return lax.axis_index("s") * NCORE + lax.axis_index("c")


def _leaky(x):
    return jnp.where(x >= 0, x, NEG_SLOPE * x)


_K1_CH = 64
_K1_ROWS = NP // NW


@functools.partial(
    pl.kernel,
    out_type=jax.ShapeDtypeStruct((NP, D), jnp.float32),
    mesh=_mesh,
    compiler_params=_sc_params,
    scratch_types=[
        pltpu.VMEM((_K1_CH,), jnp.int32),
        pltpu.VMEM((_K1_CH, D), jnp.float32),
    ],
)
def _sc_gather_feat(emb_hbm, codes_hbm, out_hbm, idx_v, rows_v):
    base = _wid() * _K1_ROWS

    def chunk(k, c):
        off = base + k * _K1_CH
        pltpu.sync_copy(codes_hbm.at[pl.ds(off, _K1_CH)], idx_v)
        pltpu.sync_copy(emb_hbm.at[idx_v], rows_v)
        pltpu.sync_copy(rows_v, out_hbm.at[pl.ds(off, _K1_CH)])
        return c

    lax.fori_loop(0, _K1_ROWS // _K1_CH, chunk, 0)


_BN = 512


def _tc1_body(feat_ref, wea_ref, bea_ref, ah_ref, at_ref, w_ref,
              rel_emb_ref, wra_ref, bra_ref, ar_ref,
              t_ref, ehet_ref, rel_ref):
    i = pl.program_id(0)
    feat = feat_ref[...]
    mh = feat @ wea_ref[...] + bea_ref[...]
    for h in range(H):
        mh_h = mh[:, h * D:(h + 1) * D]
        ehet_ref[h, :] = (mh_h * ah_ref[h:h + 1, :]).sum(axis=1)
        ehet_ref[H + h, :] = (mh_h * at_ref[h:h + 1, :]).sum(axis=1)
        for r in range(R):
            t_ref[r, :, h * D:(h + 1) * D] = jnp.dot(
                mh_h, w_ref[r],
                preferred_element_type=jnp.float32).astype(jnp.bfloat16)

    @pl.when(i == 0)
    def _():
        rh = rel_emb_ref[...] @ wra_ref[...] + bra_ref[...]
        cols = []
        for h in range(H):
            c = (rh[:, h * D:(h + 1) * D] * ar_ref[h:h + 1, :]).sum(axis=1)
            cols.append(c.reshape(R, 1))
        er = _leaky(jnp.concatenate(cols, axis=1))
        ex = jnp.exp(er - er.max(axis=0, keepdims=True))
        rel_ref[...] = ex / ex.sum(axis=0, keepdims=True)


def _tc1(feat, wea, bea, ah, at, w, rel_emb, wra, bra, ar):
    grid = NP // _BN
    return pl.pallas_call(
        _tc1_body,
        grid=(grid,),
        in_specs=[
            pl.BlockSpec((_BN, D), lambda i: (i, 0)),
            pl.BlockSpec((D, H * D), lambda i: (0, 0)),
            pl.BlockSpec((1, H * D), lambda i: (0, 0)),
            pl.BlockSpec((H, D), lambda i: (0, 0)),
            pl.BlockSpec((H, D), lambda i: (0, 0)),
            pl.BlockSpec((R, D, D), lambda i: (0, 0, 0)),
            pl.BlockSpec((R, D), lambda i: (0, 0)),
            pl.BlockSpec((D, H * D), lambda i: (0, 0)),
            pl.BlockSpec((1, H * D), lambda i: (0, 0)),
            pl.BlockSpec((H, D), lambda i: (0, 0)),
        ],
        out_specs=[
            pl.BlockSpec((R, _BN, H * D), lambda i: (0, i, 0)),
            pl.BlockSpec((8, _BN), lambda i: (0, i)),
            pl.BlockSpec((R, H), lambda i: (0, 0)),
        ],
        out_shape=[
            jax.ShapeDtypeStruct((R, NP, H * D), jnp.bfloat16),
            jax.ShapeDtypeStruct((8, NP), jnp.float32),
            jax.ShapeDtypeStruct((R, H), jnp.float32),
        ],
    )(feat, wea, bea, ah, at, w, rel_emb, wra, bra, ar)


_CH = 128
_NCH = EW // _CH
_SSTRIPE = NP * H // NSUB


@functools.partial(
    pl.kernel,
    out_type=(
        jax.ShapeDtypeStruct((EPAD * H,), jnp.float32),
        jax.ShapeDtypeStruct((NCORE, NP * H), jnp.float32),
    ),
    mesh=_mesh,
    compiler_params=_sc_params,
    scratch_types=[
        pltpu.VMEM((8 * NP,), jnp.float32),
        pltpu.VMEM((_CH,), jnp.int32),
        pltpu.VMEM((_CH,), jnp.int32),
        pltpu.VMEM((_CH * H,), jnp.float32),
        pltpu.VMEM((_CH * H,), jnp.int32),
        pltpu.VMEM((_SSTRIPE,), jnp.float32),
        pltpu.VMEM_SHARED((NP * H,), jnp.float32),
    ],
)
def _sc_k2a(ehet_hbm, src_hbm, dst_hbm, e2_hbm, s_hbm,
            ehet_v, srcc, dstc, e2c, sidxc, stage, s_sp):
    cid = lax.axis_index("c")
    sid = lax.axis_index("s")
    ebase = _wid() * EW

    def zs(i, c):
        stage[pl.ds(i * L, L)] = jnp.zeros((L,), jnp.float32)
        return c
    lax.fori_loop(0, _SSTRIPE // L, zs, 0)
    pltpu.sync_copy(stage, s_sp.at[pl.ds(sid * _SSTRIPE, _SSTRIPE)])

    pltpu.sync_copy(ehet_hbm, ehet_v)
    plsc.subcore_barrier()

    lanes = lax.iota(jnp.int32, L)

    def chunk(k, c):
        off = ebase + k * _CH
        pltpu.sync_copy(src_hbm.at[pl.ds(off, _CH)], srcc)
        pltpu.sync_copy(dst_hbm.at[pl.ds(off, _CH)], dstc)

        def grp(j, c2):
            srcv = srcc[pl.ds(j * L, L)]
            dstv = dstc[pl.ds(j * L, L)]
            ce = lanes + j * L
            for h in range(H):
                ehs = plsc.load_gather(ehet_v, [srcv + h * NP])
                etd = plsc.load_gather(ehet_v, [dstv + (H + h) * NP])
                e2 = jnp.exp(_leaky(ehs + etd))
                plsc.store_scatter(e2c, [ce * H + h], e2)
                plsc.store_scatter(sidxc, [ce * H + h], dstv * H + h)
            return c2
        lax.fori_loop(0, _CH // L, grp, 0)
        pltpu.sync_copy(e2c, e2_hbm.at[pl.ds(off * H, _CH * H)])
        pltpu.sync_copy(e2c, s_sp.at[sidxc], add=True)
        return c

    lax.fori_loop(0, _NCH, chunk, 0)
    plsc.subcore_barrier()
    pltpu.sync_copy(s_sp.at[pl.ds(sid * _SSTRIPE, _SSTRIPE)], stage)
    pltpu.sync_copy(stage, s_hbm.at[cid].at[pl.ds(sid * _SSTRIPE, _SSTRIPE)])


@functools.partial(
    pl.kernel,
    out_type=(
        jax.ShapeDtypeStruct((EPAD * H,), jnp.float32),
        jax.ShapeDtypeStruct((EPAD * H,), jnp.float32),
        jax.ShapeDtypeStruct((EPAD * H,), jnp.float32),
    ),
    mesh=_mesh,
    compiler_params=_sc_params,
    scratch_types=[
        pltpu.VMEM((NP * H,), jnp.float32),
        pltpu.VMEM((NP * H,), jnp.float32),
        pltpu.VMEM((EW,), jnp.int32),
        pltpu.VMEM((EW,), jnp.int32),
        pltpu.VMEM((R * H,), jnp.float32),
        pltpu.VMEM((_CH * H,), jnp.float32),
        pltpu.VMEM((_CH * H,), jnp.float32),
        pltpu.VMEM((_CH * H,), jnp.float32),
        pltpu.VMEM((_CH * H,), jnp.float32),
    ],
)
def _sc_k2b(s_hbm, e2_hbm, dst_hbm, ety_hbm, rel_hbm,
            ea_hbm, re_hbm, att_hbm,
            s0_v, s1_v, dst_v, ety_v, rel_v, e2c, eac, rec, attc):
    ebase = _wid() * EW

    pltpu.sync_copy(s_hbm.at[0], s0_v)
    pltpu.sync_copy(s_hbm.at[1], s1_v)
    pltpu.sync_copy(dst_hbm.at[pl.ds(ebase, EW)], dst_v)
    pltpu.sync_copy(ety_hbm.at[pl.ds(ebase, EW)], ety_v)
    pltpu.sync_copy(rel_hbm, rel_v)

    def addp(i, c):
        s0_v[pl.ds(i * L, L)] = s0_v[pl.ds(i * L, L)] + s1_v[pl.ds(i * L, L)]
        return c
    lax.fori_loop(0, NP * H // L, addp, 0)

    lanes = lax.iota(jnp.int32, L)
    hsub = jnp.bitwise_and(lanes, H - 1)

    def chunk(k, c):
        pltpu.sync_copy(e2_hbm.at[pl.ds((ebase + k * _CH) * H, _CH * H)], e2c)

        def grp(j2, c2):
            p0 = j2 * L
            ce = k * _CH + jnp.right_shift(lanes + p0, 2)
            e2 = e2c[pl.ds(p0, L)]
            dstv = plsc.load_gather(dst_v, [ce])
            etyv = plsc.load_gather(ety_v, [ce])
            sv = plsc.load_gather(s0_v, [dstv * H + hsub])
            ea = e2 / (sv + 1e-9)
            rv = plsc.load_gather(rel_v, [etyv * H + hsub])
            eac[pl.ds(p0, L)] = ea
            rec[pl.ds(p0, L)] = rv
            attc[pl.ds(p0, L)] = ea * rv
            return c2
        lax.fori_loop(0, _CH * H // L, grp, 0)
        off = (ebase + k * _CH) * H
        pltpu.sync_copy(eac, ea_hbm.at[pl.ds(off, _CH * H)])
        pltpu.sync_copy(rec, re_hbm.at[pl.ds(off, _CH * H)])
        pltpu.sync_copy(attc, att_hbm.at[pl.ds(off, _CH * H)])
        return c

    lax.fori_loop(0, _NCH, chunk, 0)


_CH3 = 16
_SCE = 512
_KPS = _SCE // _CH3
_NSC = EW // _SCE
_NBUF = 4
_AROWS = NP // NSUB
_XCH = 16


@functools.partial(
    pl.kernel,
    out_type=jax.ShapeDtypeStruct((NCORE, NP, D), jnp.float32),
    mesh=_mesh,
    compiler_params=_sc_params,
    scratch_types=[
        pltpu.VMEM((_SCE,), jnp.int32),
        pltpu.VMEM((_SCE,), jnp.int32),
        pltpu.VMEM((_SCE,), jnp.int32),
        pltpu.VMEM((_SCE * H,), jnp.float32),
        pltpu.VMEM((_CH3,), jnp.int32),
        pltpu.VMEM((_CH3,), jnp.int32),
        pltpu.VMEM((_CH3,), jnp.int32),
        pltpu.VMEM((_CH3,), jnp.int32),
        pltpu.VMEM((_CH3,), jnp.int32),
        pltpu.VMEM((_CH3,), jnp.int32),
        pltpu.VMEM((_CH3,), jnp.int32),
        pltpu.VMEM((_CH3,), jnp.int32),
        pltpu.VMEM((_CH3, H * D // 2), jnp.int32),
        pltpu.VMEM((_CH3, H * D // 2), jnp.int32),
        pltpu.VMEM((_CH3, H * D // 2), jnp.int32),
        pltpu.VMEM((_CH3, H * D // 2), jnp.int32),
        pltpu.VMEM((_CH3, D), jnp.float32),
        pltpu.VMEM((_CH3, D), jnp.float32),
        pltpu.VMEM((_CH3, D), jnp.float32),
        pltpu.VMEM((_CH3, D), jnp.float32),
        pltpu.SemaphoreType.DMA,
        pltpu.SemaphoreType.DMA,
        pltpu.SemaphoreType.DMA,
        pltpu.SemaphoreType.DMA,
        pltpu.SemaphoreType.DMA,
        pltpu.SemaphoreType.DMA,
        pltpu.SemaphoreType.DMA,
        pltpu.SemaphoreType.DMA,
        pltpu.VMEM_SHARED((NP, D), jnp.float32),
    ],
)
def _sc_k3(t_hbm, src_hbm, ety_hbm, dst_hbm, att_hbm, agg_hbm,
           srcs, etys, dsts, atts, gidx0, gidx1, gidx2, gidx3,
           sidx0, sidx1, sidx2, sidx3, rows0, rows1, rows2, rows3,
           v0, v1, v2, v3, gsem0, gsem1, gsem2, gsem3,
           ssem0, ssem1, ssem2, ssem3, agg_sp):
    cid = lax.axis_index("c")
    sid = lax.axis_index("s")
    ebase = _wid() * EW
    gidx = (gidx0, gidx1, gidx2, gidx3)
    sidx = (sidx0, sidx1, sidx2, sidx3)
    rows = (rows0, rows1, rows2, rows3)
    vv = (v0, v1, v2, v3)
    gsem = (gsem0, gsem1, gsem2, gsem3)
    ssem = (ssem0, ssem1, ssem2, ssem3)

    def zrow(e, c):
        for cc in range(D // L):
            v0[e, pl.ds(cc * L, L)] = jnp.zeros((L,), jnp.float32)
        return c
    lax.fori_loop(0, _XCH, zrow, 0)

    def zc(i, c):
        pltpu.sync_copy(
            v0, agg_sp.at[pl.ds(sid * _AROWS + i * _XCH, _XCH)])
        return c
    lax.fori_loop(0, _AROWS // _XCH, zc, 0)
    plsc.subcore_barrier()

    def build_g(k, b):
        srcv = srcs[pl.ds(k * _CH3, L)]
        etyv = etys[pl.ds(k * _CH3, L)]
        gidx[b][...] = etyv * NP + srcv

    def compute(k, b):
        abase = jnp.full((L,), k * (_CH3 * H), jnp.int32)
        himask = jnp.full((L,), -65536, jnp.int32)
        for e in range(_CH3):
            av = [plsc.load_gather(atts, [abase + (e * H + h)])
                  for h in range(H)]
            for s in range(D // 32):
                acc_e = acc_o = None
                for h in range(H):
                    w = rows[b][e, pl.ds(h * (D // 2) + s * L, L)]
                    fe = plsc.bitcast(jnp.left_shift(w, 16), jnp.float32)
                    fo = plsc.bitcast(
                        jnp.bitwise_and(w, himask), jnp.float32)
                    if h == 0:
                        acc_e = av[0] * fe
                        acc_o = av[0] * fo
                    else:
                        acc_e = acc_e + av[h] * fe
                        acc_o = acc_o + av[h] * fo
                vv[b][e, pl.ds(s * 32, L)] = acc_e
                vv[b][e, pl.ds(s * 32 + L, L)] = acc_o

    def chunk_body(p, k, b):
        pltpu.make_async_copy(t_hbm.at[gidx[b]], rows[b], gsem[b]).wait()
        pltpu.make_async_copy(vv[b], agg_sp.at[sidx[b]], ssem[b]).wait()
        sidx[b][...] = dsts[pl.ds(k * _CH3, L)]
        compute(k, b)
        pltpu.async_copy(vv[b], agg_sp.at[sidx[b]], ssem[b], add=True)

        @pl.when(p < _KPS // _NBUF - 1)
        def _():
            build_g(k + _NBUF, b)
            pltpu.async_copy(t_hbm.at[gidx[b]], rows[b], gsem[b])

    def superchunk(s, c):
        off = ebase + s * _SCE
        pltpu.sync_copy(src_hbm.at[pl.ds(off, _SCE)], srcs)
        pltpu.sync_copy(ety_hbm.at[pl.ds(off, _SCE)], etys)
        pltpu.sync_copy(dst_hbm.at[pl.ds(off, _SCE)], dsts)
        pltpu.sync_copy(att_hbm.at[pl.ds(off * H, _SCE * H)], atts)

        for b in range(_NBUF):
            sidx[b][...] = jnp.full((L,), DUMMY_DST, jnp.int32)
            pltpu.async_copy(vv[b], agg_sp.at[sidx[b]], ssem[b], add=True)
            build_g(b, b)
            pltpu.async_copy(t_hbm.at[gidx[b]], rows[b], gsem[b])

        def quad(p, c2):
            for b in range(_NBUF):
                chunk_body(p, _NBUF * p + b, b)
            return c2
        lax.fori_loop(0, _KPS // _NBUF, quad, 0)

        for b in range(_NBUF):
            pltpu.make_async_copy(vv[b], agg_sp.at[sidx[b]], ssem[b]).wait()
        return c

    lax.fori_loop(0, _NSC, superchunk, 0)
    plsc.subcore_barrier()

    def xc(i, c):
        r0 = sid * _AROWS + i * _XCH
        pltpu.sync_copy(agg_sp.at[pl.ds(r0, _XCH)], v0)
        pltpu.sync_copy(v0, agg_hbm.at[cid].at[pl.ds(r0, _XCH)])
        return c
    lax.fori_loop(0, _AROWS // _XCH, xc, 0)


_BN2 = 1024


def _tc2_body(a0_ref, a1_ref, feat_ref, bias_ref, out_ref):
    agg = a0_ref[...] + a1_ref[...] + bias_ref[...]
    out_ref[...] = jnp.maximum(
        (1.0 - LOOP_LAMBDA) * agg + LOOP_LAMBDA * feat_ref[...], 0.0)


def _tc2(a0, a1, feat, bias):
    return pl.pallas_call(
        _tc2_body,
        grid=(NP // _BN2,),
        in_specs=[
            pl.BlockSpec((_BN2, D), lambda i: (i, 0)),
            pl.BlockSpec((_BN2, D), lambda i: (i, 0)),
            pl.BlockSpec((_BN2, D), lambda i: (i, 0)),
            pl.BlockSpec((1, D), lambda i: (0, 0)),
        ],
        out_specs=pl.BlockSpec((_BN2, D), lambda i: (i, 0)),
        out_shape=jax.ShapeDtypeStruct((NP, D), jnp.float32),
    )(a0, a1, feat, bias)


def kernel(node_codes, edge_index, etypes, code_emb, rel_emb, W, h_bias,
           W_edge_att, b_edge_att, W_rel_att, b_rel_att,
           attn_h, attn_t, attn_r):
    i32 = jnp.int32
    codes_p = jnp.pad(node_codes.astype(i32), (0, NP - N))
    src_p = jnp.pad(edge_index[0].astype(i32), (0, EPAD - E))
    dst_p = jnp.pad(edge_index[1].astype(i32), (0, EPAD - E),
                    constant_values=DUMMY_DST)
    ety_p = jnp.pad(etypes.astype(i32), (0, EPAD - E))

    feat = _sc_gather_feat(code_emb, codes_p)

    t, ehet, rel_att = _tc1(
        feat, W_edge_att, b_edge_att.reshape(1, H * D),
        attn_h.reshape(H, D), attn_t.reshape(H, D), W[:, :, _TPERM],
        rel_emb, W_rel_att, b_rel_att.reshape(1, H * D),
        attn_r.reshape(H, D))
    t_flat = jax.lax.bitcast_convert_type(
        t.reshape(R * NP, H * D // 2, 2), jnp.int32)

    e2, s_parts = _sc_k2a(ehet.reshape(8 * NP), src_p, dst_p)
    ea_f, re_f, att_f = _sc_k2b(
        s_parts, e2, dst_p, ety_p, rel_att.reshape(R * H))

    agg = _sc_k3(t_flat, src_p, ety_p, dst_p, att_f)

    node_repr = _tc2(agg[0], agg[1], feat, h_bias.reshape(1, D))
    edge_att = ea_f.reshape(EPAD, H)[:E]
    rel_att_e = re_f.reshape(EPAD, H)[:E]
    return node_repr[:N], rel_att_e, edge_att


# speedup vs baseline: 1.4632x; 1.4632x over previous
import functools

import jax
import jax.numpy as jnp
from jax import lax
from jax.experimental import pallas as pl
from jax.experimental.pallas import tpu as pltpu
from jax.experimental.pallas import tpu_sc as plsc

N = 10000
E = 160000
D = 128
H = 4
R = 4
LOOP_LAMBDA = 0.5
NEG_SLOPE = 0.2

NCORE = 2
NSUB = 16
NW = NCORE * NSUB
L = 16

NP = 10240
EW = 5120
EPAD = NW * EW
DUMMY_DST = N + 200

_mesh = plsc.VectorSubcoreMesh(
    core_axis_name="c", subcore_axis_name="s", num_cores=NCORE,
    num_subcores=NSUB)
_sc_params = pltpu.CompilerParams(needs_layout_passes=False)


def _wid():
    return lax.axis_index("s") * NCORE + lax.axis_index("c")


def _leaky(x):
    return jnp.where(x >= 0, x, NEG_SLOPE * x)


_K1_CH = 64
_K1_ROWS = NP // NW


@functools.partial(
    pl.kernel,
    out_type=jax.ShapeDtypeStruct((NP, D), jnp.float32),
    mesh=_mesh,
    compiler_params=_sc_params,
    scratch_types=[
        pltpu.VMEM((_K1_CH,), jnp.int32),
        pltpu.VMEM((_K1_CH, D), jnp.float32),
    ],
)
def _sc_gather_feat(emb_hbm, codes_hbm, out_hbm, idx_v, rows_v):
    base = _wid() * _K1_ROWS

    def chunk(k, c):
        off = base + k * _K1_CH
        pltpu.sync_copy(codes_hbm.at[pl.ds(off, _K1_CH)], idx_v)
        pltpu.sync_copy(emb_hbm.at[idx_v], rows_v)
        pltpu.sync_copy(rows_v, out_hbm.at[pl.ds(off, _K1_CH)])
        return c

    lax.fori_loop(0, _K1_ROWS // _K1_CH, chunk, 0)


_BN = 512


def _tc1_body(feat_ref, wea_ref, bea_ref, ah_ref, at_ref, w_ref,
              rel_emb_ref, wra_ref, bra_ref, ar_ref,
              t_ref, ehet_ref, rel_ref):
    i = pl.program_id(0)
    feat = feat_ref[...]
    mh = feat @ wea_ref[...] + bea_ref[...]
    for h in range(H):
        mh_h = mh[:, h * D:(h + 1) * D]
        ehet_ref[h, :] = (mh_h * ah_ref[h:h + 1, :]).sum(axis=1)
        ehet_ref[H + h, :] = (mh_h * at_ref[h:h + 1, :]).sum(axis=1)
        for r in range(R):
            t_ref[r, :, h * D:(h + 1) * D] = jnp.dot(
                mh_h, w_ref[r], preferred_element_type=jnp.float32)

    @pl.when(i == 0)
    def _():
        rh = rel_emb_ref[...] @ wra_ref[...] + bra_ref[...]
        cols = []
        for h in range(H):
            c = (rh[:, h * D:(h + 1) * D] * ar_ref[h:h + 1, :]).sum(axis=1)
            cols.append(c.reshape(R, 1))
        er = _leaky(jnp.concatenate(cols, axis=1))
        ex = jnp.exp(er - er.max(axis=0, keepdims=True))
        rel_ref[...] = ex / ex.sum(axis=0, keepdims=True)


def _tc1(feat, wea, bea, ah, at, w, rel_emb, wra, bra, ar):
    grid = NP // _BN
    return pl.pallas_call(
        _tc1_body,
        grid=(grid,),
        in_specs=[
            pl.BlockSpec((_BN, D), lambda i: (i, 0)),
            pl.BlockSpec((D, H * D), lambda i: (0, 0)),
            pl.BlockSpec((1, H * D), lambda i: (0, 0)),
            pl.BlockSpec((H, D), lambda i: (0, 0)),
            pl.BlockSpec((H, D), lambda i: (0, 0)),
            pl.BlockSpec((R, D, D), lambda i: (0, 0, 0)),
            pl.BlockSpec((R, D), lambda i: (0, 0)),
            pl.BlockSpec((D, H * D), lambda i: (0, 0)),
            pl.BlockSpec((1, H * D), lambda i: (0, 0)),
            pl.BlockSpec((H, D), lambda i: (0, 0)),
        ],
        out_specs=[
            pl.BlockSpec((R, _BN, H * D), lambda i: (0, i, 0)),
            pl.BlockSpec((8, _BN), lambda i: (0, i)),
            pl.BlockSpec((R, H), lambda i: (0, 0)),
        ],
        out_shape=[
            jax.ShapeDtypeStruct((R, NP, H * D), jnp.float32),
            jax.ShapeDtypeStruct((8, NP), jnp.float32),
            jax.ShapeDtypeStruct((R, H), jnp.float32),
        ],
    )(feat, wea, bea, ah, at, w, rel_emb, wra, bra, ar)


_CH = 128
_NCH = EW // _CH
_SSTRIPE = NP * H // NSUB


@functools.partial(
    pl.kernel,
    out_type=(
        jax.ShapeDtypeStruct((EPAD * H,), jnp.float32),
        jax.ShapeDtypeStruct((NCORE, NP * H), jnp.float32),
    ),
    mesh=_mesh,
    compiler_params=_sc_params,
    scratch_types=[
        pltpu.VMEM((8 * NP,), jnp.float32),
        pltpu.VMEM((EW,), jnp.int32),
        pltpu.VMEM((EW,), jnp.int32),
        pltpu.VMEM((_CH * H,), jnp.float32),
        pltpu.VMEM((_CH * H,), jnp.float32),
        pltpu.VMEM((_CH * H,), jnp.int32),
        pltpu.VMEM((_CH * H,), jnp.int32),
        pltpu.VMEM((_SSTRIPE,), jnp.float32),
        pltpu.SemaphoreType.DMA,
        pltpu.SemaphoreType.DMA,
        pltpu.SemaphoreType.DMA,
        pltpu.SemaphoreType.DMA,
        pltpu.VMEM_SHARED((NP * H,), jnp.float32),
    ],
)
def _sc_k2a(ehet_hbm, src_hbm, dst_hbm, e2_hbm, s_hbm,
            ehet_v, src_v, dst_v, e2c0, e2c1, sx0, sx1, stage,
            esem0, esem1, ssem0, ssem1, s_sp):
    cid = lax.axis_index("c")
    sid = lax.axis_index("s")
    ebase = _wid() * EW
    e2c = (e2c0, e2c1)
    sx = (sx0, sx1)
    esem = (esem0, esem1)
    ssem = (ssem0, ssem1)

    def zs(i, c):
        stage[pl.ds(i * L, L)] = jnp.zeros((L,), jnp.float32)
        return c
    lax.fori_loop(0, _SSTRIPE // L, zs, 0)
    pltpu.sync_copy(stage, s_sp.at[pl.ds(sid * _SSTRIPE, _SSTRIPE)])

    pltpu.sync_copy(ehet_hbm, ehet_v)
    pltpu.sync_copy(src_hbm.at[pl.ds(ebase, EW)], src_v)
    pltpu.sync_copy(dst_hbm.at[pl.ds(ebase, EW)], dst_v)
    plsc.subcore_barrier()

    lanes = lax.iota(jnp.int32, L)

    def chunk_body(p, k, b):
        @pl.when(p >= 1)
        def _():
            pltpu.make_async_copy(
                e2c[b], e2_hbm.at[pl.ds(0, _CH * H)], esem[b]).wait()
            pltpu.make_async_copy(e2c[b], s_sp.at[sx[b]], ssem[b]).wait()

        def grp(j, c2):
            o = k * _CH + j * L
            srcv = src_v[pl.ds(o, L)]
            dstv = dst_v[pl.ds(o, L)]
            ce = lanes + j * L
            for h in range(H):
                ehs = plsc.load_gather(ehet_v, [srcv + h * NP])
                etd = plsc.load_gather(ehet_v, [dstv + (H + h) * NP])
                e2 = jnp.exp(_leaky(ehs + etd))
                plsc.store_scatter(e2c[b], [ce * H + h], e2)
                plsc.store_scatter(sx[b], [ce * H + h], dstv * H + h)
            return c2
        lax.fori_loop(0, _CH // L, grp, 0)
        pltpu.async_copy(
            e2c[b], e2_hbm.at[pl.ds((ebase + k * _CH) * H, _CH * H)],
            esem[b])
        pltpu.async_copy(e2c[b], s_sp.at[sx[b]], ssem[b], add=True)

    def pair(p, c):
        chunk_body(p, 2 * p, 0)
        chunk_body(p, 2 * p + 1, 1)
        return c
    lax.fori_loop(0, _NCH // 2, pair, 0)

    for b in range(2):
        pltpu.make_async_copy(
            e2c[b], e2_hbm.at[pl.ds(0, _CH * H)], esem[b]).wait()
        pltpu.make_async_copy(e2c[b], s_sp.at[sx[b]], ssem[b]).wait()
    plsc.subcore_barrier()
    pltpu.sync_copy(s_sp.at[pl.ds(sid * _SSTRIPE, _SSTRIPE)], stage)
    pltpu.sync_copy(stage, s_hbm.at[cid].at[pl.ds(sid * _SSTRIPE, _SSTRIPE)])


@functools.partial(
    pl.kernel,
    out_type=(
        jax.ShapeDtypeStruct((EPAD * H,), jnp.float32),
        jax.ShapeDtypeStruct((EPAD * H,), jnp.float32),
        jax.ShapeDtypeStruct((EPAD * H,), jnp.float32),
    ),
    mesh=_mesh,
    compiler_params=_sc_params,
    scratch_types=[
        pltpu.VMEM((NP * H,), jnp.float32),
        pltpu.VMEM((NP * H,), jnp.float32),
        pltpu.VMEM((EW,), jnp.int32),
        pltpu.VMEM((EW,), jnp.int32),
        pltpu.VMEM((R * H,), jnp.float32),
        pltpu.VMEM((_CH * H,), jnp.float32),
        pltpu.VMEM((_CH * H,), jnp.float32),
        pltpu.VMEM((_CH * H,), jnp.float32),
        pltpu.VMEM((_CH * H,), jnp.float32),
        pltpu.VMEM((_CH * H,), jnp.float32),
        pltpu.VMEM((_CH * H,), jnp.float32),
        pltpu.VMEM((_CH * H,), jnp.float32),
        pltpu.VMEM((_CH * H,), jnp.float32),
        pltpu.SemaphoreType.DMA,
        pltpu.SemaphoreType.DMA,
        pltpu.SemaphoreType.DMA,
        pltpu.SemaphoreType.DMA,
    ],
)
def _sc_k2b(s_hbm, e2_hbm, dst_hbm, ety_hbm, rel_hbm,
            ea_hbm, re_hbm, att_hbm,
            s0_v, s1_v, dst_v, ety_v, rel_v, e2c0, e2c1,
            eac0, eac1, rec0, rec1, attc0, attc1,
            lsem0, lsem1, osem0, osem1):
    ebase = _wid() * EW
    e2c = (e2c0, e2c1)
    eac = (eac0, eac1)
    rec = (rec0, rec1)
    attc = (attc0, attc1)
    lsem = (lsem0, lsem1)
    osem = (osem0, osem1)

    pltpu.sync_copy(s_hbm.at[0], s0_v)
    pltpu.sync_copy(s_hbm.at[1], s1_v)
    pltpu.sync_copy(dst_hbm.at[pl.ds(ebase, EW)], dst_v)
    pltpu.sync_copy(ety_hbm.at[pl.ds(ebase, EW)], ety_v)
    pltpu.sync_copy(rel_hbm, rel_v)

    def addp(i, c):
        s0_v[pl.ds(i * L, L)] = s0_v[pl.ds(i * L, L)] + s1_v[pl.ds(i * L, L)]
        return c
    lax.fori_loop(0, NP * H // L, addp, 0)

    lanes = lax.iota(jnp.int32, L)
    hsub = jnp.bitwise_and(lanes, H - 1)

    for b in range(2):
        pltpu.async_copy(
            e2_hbm.at[pl.ds((ebase + b * _CH) * H, _CH * H)], e2c[b],
            lsem[b])

    def chunk_body(p, k, b):
        pltpu.make_async_copy(
            e2_hbm.at[pl.ds(0, _CH * H)], e2c[b], lsem[b]).wait()

        @pl.when(p >= 1)
        def _():
            for ref in (eac[b], rec[b], attc[b]):
                pltpu.make_async_copy(
                    ref, ea_hbm.at[pl.ds(0, _CH * H)], osem[b]).wait()

        def grp(j2, c2):
            p0 = j2 * L
            ce = k * _CH + jnp.right_shift(lanes + p0, 2)
            e2 = e2c[b][pl.ds(p0, L)]
            dstv = plsc.load_gather(dst_v, [ce])
            etyv = plsc.load_gather(ety_v, [ce])
            sv = plsc.load_gather(s0_v, [dstv * H + hsub])
            ea = e2 / (sv + 1e-9)
            rv = plsc.load_gather(rel_v, [etyv * H + hsub])
            eac[b][pl.ds(p0, L)] = ea
            rec[b][pl.ds(p0, L)] = rv
            attc[b][pl.ds(p0, L)] = ea * rv
            return c2
        lax.fori_loop(0, _CH * H // L, grp, 0)
        off = (ebase + k * _CH) * H
        pltpu.async_copy(eac[b], ea_hbm.at[pl.ds(off, _CH * H)], osem[b])
        pltpu.async_copy(rec[b], re_hbm.at[pl.ds(off, _CH * H)], osem[b])
        pltpu.async_copy(attc[b], att_hbm.at[pl.ds(off, _CH * H)], osem[b])

        @pl.when(p < _NCH // 2 - 1)
        def _():
            pltpu.async_copy(
                e2_hbm.at[pl.ds((ebase + (k + 2) * _CH) * H, _CH * H)],
                e2c[b], lsem[b])

    def pair(p, c):
        chunk_body(p, 2 * p, 0)
        chunk_body(p, 2 * p + 1, 1)
        return c
    lax.fori_loop(0, _NCH // 2, pair, 0)

    for b in range(2):
        for ref in (eac[b], rec[b], attc[b]):
            pltpu.make_async_copy(
                ref, ea_hbm.at[pl.ds(0, _CH * H)], osem[b]).wait()


_CH3 = 16
_SCE = 512
_KPS = _SCE // _CH3
_NSC = EW // _SCE
_NBUF = 4
_AROWS = NP // NSUB
_XCH = 16


@functools.partial(
    pl.kernel,
    out_type=jax.ShapeDtypeStruct((NCORE, NP, D), jnp.float32),
    mesh=_mesh,
    compiler_params=_sc_params,
    scratch_types=[
        pltpu.VMEM((_SCE,), jnp.int32),
        pltpu.VMEM((_SCE,), jnp.int32),
        pltpu.VMEM((_SCE,), jnp.int32),
        pltpu.VMEM((_SCE * H,), jnp.float32),
        pltpu.VMEM((_CH3,), jnp.int32),
        pltpu.VMEM((_CH3,), jnp.int32),
        pltpu.VMEM((_CH3,), jnp.int32),
        pltpu.VMEM((_CH3,), jnp.int32),
        pltpu.VMEM((_CH3,), jnp.int32),
        pltpu.VMEM((_CH3,), jnp.int32),
        pltpu.VMEM((_CH3,), jnp.int32),
        pltpu.VMEM((_CH3,), jnp.int32),
        pltpu.VMEM((_CH3, H * D), jnp.float32),
        pltpu.VMEM((_CH3, H * D), jnp.float32),
        pltpu.VMEM((_CH3, H * D), jnp.float32),
        pltpu.VMEM((_CH3, H * D), jnp.float32),
        pltpu.VMEM((_CH3, D), jnp.float32),
        pltpu.VMEM((_CH3, D), jnp.float32),
        pltpu.VMEM((_CH3, D), jnp.float32),
        pltpu.VMEM((_CH3, D), jnp.float32),
        pltpu.SemaphoreType.DMA,
        pltpu.SemaphoreType.DMA,
        pltpu.SemaphoreType.DMA,
        pltpu.SemaphoreType.DMA,
        pltpu.SemaphoreType.DMA,
        pltpu.SemaphoreType.DMA,
        pltpu.SemaphoreType.DMA,
        pltpu.SemaphoreType.DMA,
        pltpu.VMEM_SHARED((NP, D), jnp.float32),
    ],
)
def _sc_k3(t_hbm, src_hbm, ety_hbm, dst_hbm, att_hbm, agg_hbm,
           srcs, etys, dsts, atts, gidx0, gidx1, gidx2, gidx3,
           sidx0, sidx1, sidx2, sidx3, rows0, rows1, rows2, rows3,
           v0, v1, v2, v3, gsem0, gsem1, gsem2, gsem3,
           ssem0, ssem1, ssem2, ssem3, agg_sp):
    cid = lax.axis_index("c")
    sid = lax.axis_index("s")
    ebase = _wid() * EW
    gidx = (gidx0, gidx1, gidx2, gidx3)
    sidx = (sidx0, sidx1, sidx2, sidx3)
    rows = (rows0, rows1, rows2, rows3)
    vv = (v0, v1, v2, v3)
    gsem = (gsem0, gsem1, gsem2, gsem3)
    ssem = (ssem0, ssem1, ssem2, ssem3)

    def zrow(e, c):
        for cc in range(D // L):
            v0[e, pl.ds(cc * L, L)] = jnp.zeros((L,), jnp.float32)
        return c
    lax.fori_loop(0, _XCH, zrow, 0)

    def zc(i, c):
        pltpu.sync_copy(
            v0, agg_sp.at[pl.ds(sid * _AROWS + i * _XCH, _XCH)])
        return c
    lax.fori_loop(0, _AROWS // _XCH, zc, 0)
    plsc.subcore_barrier()

    def build_g(k, b):
        srcv = srcs[pl.ds(k * _CH3, L)]
        etyv = etys[pl.ds(k * _CH3, L)]
        gidx[b][...] = etyv * NP + srcv

    def compute(k, b):
        abase = jnp.full((L,), k * (_CH3 * H), jnp.int32)
        for e in range(_CH3):
            av = [plsc.load_gather(atts, [abase + (e * H + h)])
                  for h in range(H)]
            for oo in range(D // L):
                acc = av[0] * rows[b][e, pl.ds(oo * L, L)]
                for h in range(1, H):
                    acc = acc + av[h] * rows[b][e, pl.ds(h * D + oo * L, L)]
                vv[b][e, pl.ds(oo * L, L)] = acc

    def chunk_body(p, k, b):
        pltpu.make_async_copy(t_hbm.at[gidx[b]], rows[b], gsem[b]).wait()
        pltpu.make_async_copy(vv[b], agg_sp.at[sidx[b]], ssem[b]).wait()
        sidx[b][...] = dsts[pl.ds(k * _CH3, L)]
        compute(k, b)
        pltpu.async_copy(vv[b], agg_sp.at[sidx[b]], ssem[b], add=True)

        @pl.when(p < _KPS // _NBUF - 1)
        def _():
            build_g(k + _NBUF, b)
            pltpu.async_copy(t_hbm.at[gidx[b]], rows[b], gsem[b])

    def superchunk(s, c):
        off = ebase + s * _SCE
        pltpu.sync_copy(src_hbm.at[pl.ds(off, _SCE)], srcs)
        pltpu.sync_copy(ety_hbm.at[pl.ds(off, _SCE)], etys)
        pltpu.sync_copy(dst_hbm.at[pl.ds(off, _SCE)], dsts)
        pltpu.sync_copy(att_hbm.at[pl.ds(off * H, _SCE * H)], atts)

        for b in range(_NBUF):
            sidx[b][...] = jnp.full((L,), DUMMY_DST, jnp.int32)
            pltpu.async_copy(vv[b], agg_sp.at[sidx[b]], ssem[b], add=True)
            build_g(b, b)
            pltpu.async_copy(t_hbm.at[gidx[b]], rows[b], gsem[b])

        def quad(p, c2):
            for b in range(_NBUF):
                chunk_body(p, _NBUF * p + b, b)
            return c2
        lax.fori_loop(0, _KPS // _NBUF, quad, 0)

        for b in range(_NBUF):
            pltpu.make_async_copy(vv[b], agg_sp.at[sidx[b]], ssem[b]).wait()
        return c

    lax.fori_loop(0, _NSC, superchunk, 0)
    plsc.subcore_barrier()

    def xc(i, c):
        r0 = sid * _AROWS + i * _XCH
        pltpu.sync_copy(agg_sp.at[pl.ds(r0, _XCH)], v0)
        pltpu.sync_copy(v0, agg_hbm.at[cid].at[pl.ds(r0, _XCH)])
        return c
    lax.fori_loop(0, _AROWS // _XCH, xc, 0)


_BN2 = 1024


def _tc2_body(a0_ref, a1_ref, feat_ref, bias_ref, out_ref):
    agg = a0_ref[...] + a1_ref[...] + bias_ref[...]
    out_ref[...] = jnp.maximum(
        (1.0 - LOOP_LAMBDA) * agg + LOOP_LAMBDA * feat_ref[...], 0.0)


def _tc2(a0, a1, feat, bias):
    return pl.pallas_call(
        _tc2_body,
        grid=(NP // _BN2,),
        in_specs=[
            pl.BlockSpec((_BN2, D), lambda i: (i, 0)),
            pl.BlockSpec((_BN2, D), lambda i: (i, 0)),
            pl.BlockSpec((_BN2, D), lambda i: (i, 0)),
            pl.BlockSpec((1, D), lambda i: (0, 0)),
        ],
        out_specs=pl.BlockSpec((_BN2, D), lambda i: (i, 0)),
        out_shape=jax.ShapeDtypeStruct((NP, D), jnp.float32),
    )(a0, a1, feat, bias)


def kernel(node_codes, edge_index, etypes, code_emb, rel_emb, W, h_bias,
           W_edge_att, b_edge_att, W_rel_att, b_rel_att,
           attn_h, attn_t, attn_r):
    i32 = jnp.int32
    codes_p = jnp.pad(node_codes.astype(i32), (0, NP - N))
    src_p = jnp.pad(edge_index[0].astype(i32), (0, EPAD - E))
    dst_p = jnp.pad(edge_index[1].astype(i32), (0, EPAD - E),
                    constant_values=DUMMY_DST)
    ety_p = jnp.pad(etypes.astype(i32), (0, EPAD - E))

    feat = _sc_gather_feat(code_emb, codes_p)

    t, ehet, rel_att = _tc1(
        feat, W_edge_att, b_edge_att.reshape(1, H * D),
        attn_h.reshape(H, D), attn_t.reshape(H, D), W,
        rel_emb, W_rel_att, b_rel_att.reshape(1, H * D),
        attn_r.reshape(H, D))
    t_flat = t.reshape(R * NP, H * D)

    e2, s_parts = _sc_k2a(ehet.reshape(8 * NP), src_p, dst_p)
    ea_f, re_f, att_f = _sc_k2b(
        s_parts, e2, dst_p, ety_p, rel_att.reshape(R * H))

    agg = _sc_k3(t_flat, src_p, ety_p, dst_p, att_f)

    node_repr = _tc2(agg[0], agg[1], feat, h_bias.reshape(1, D))
    edge_att = ea_f.reshape(EPAD, H)[:E]
    rel_att_e = re_f.reshape(EPAD, H)[:E]
    return node_repr[:N], rel_att_e, edge_att
